# Initial kernel scaffold; baseline (speedup 1.0000x reference)
import jax
import jax.numpy as jnp
from jax.experimental import pallas as pl


def kernel(x, time_embed):
    raise NotImplementedError("write your pallas kernel here")


# speedup vs baseline: 1.7203x; 1.7203x over previous
import jax
import jax.numpy as jnp
from jax.experimental import pallas as pl


S_BLK = 256


def _add_pos_kernel(x_ref, pos_ref, o_ref):
    o_ref[...] = x_ref[...] + pos_ref[...][None, :, :]


def kernel(x, time_embed):
    B, S, d = x.shape
    grid = (S // S_BLK,)
    return pl.pallas_call(
        _add_pos_kernel,
        grid=grid,
        in_specs=[
            pl.BlockSpec((B, S_BLK, d), lambda i: (0, i, 0)),
            pl.BlockSpec((S_BLK, d), lambda i: (i, 0)),
        ],
        out_specs=pl.BlockSpec((B, S_BLK, d), lambda i: (0, i, 0)),
        out_shape=jax.ShapeDtypeStruct((B, S, d), x.dtype),
    )(x, time_embed)


# speedup vs baseline: 1.7272x; 1.0040x over previous
import jax
import jax.numpy as jnp
from jax.experimental import pallas as pl


S_BLK = 512


def _add_pos_kernel(x_ref, pos_ref, o_ref):
    o_ref[...] = x_ref[...] + pos_ref[...][None, :, :]


def kernel(x, time_embed):
    B, S, d = x.shape
    grid = (S // S_BLK,)
    return pl.pallas_call(
        _add_pos_kernel,
        grid=grid,
        in_specs=[
            pl.BlockSpec((B, S_BLK, d), lambda i: (0, i, 0)),
            pl.BlockSpec((S_BLK, d), lambda i: (i, 0)),
        ],
        out_specs=pl.BlockSpec((B, S_BLK, d), lambda i: (0, i, 0)),
        out_shape=jax.ShapeDtypeStruct((B, S, d), x.dtype),
    )(x, time_embed)
